# speedup vs baseline: 3.4176x; 3.4176x over previous
import functools

import jax
import jax.numpy as jnp
from jax.experimental import pallas as pl
from jax.experimental.pallas import tpu as pltpu

_TOP_K = 2


def _gating_block(x_ref, w_ref, gates_ref, load_ref):
    x = x_ref[...]
    w = w_ref[...]
    logits = jax.lax.dot_general(
        x, w, (((1,), (1,)), ((), ())),
        preferred_element_type=jnp.float32)

    e = logits.shape[1]
    iota = jax.lax.broadcasted_iota(jnp.int32, logits.shape, 1)

    max1 = jnp.max(logits, axis=1, keepdims=True)
    ex = jnp.exp(logits - max1)
    load_ref[...] = ex / jnp.sum(ex, axis=1, keepdims=True)

    idx1 = jnp.min(jnp.where(logits == max1, iota, e), axis=1, keepdims=True)
    is1 = iota == idx1
    masked = jnp.where(is1, -jnp.inf, logits)
    max2 = jnp.max(masked, axis=1, keepdims=True)
    idx2 = jnp.min(jnp.where(masked == max2, iota, e), axis=1, keepdims=True)

    e2 = jnp.exp(max2 - max1)
    denom = 1.0 + e2
    gates_ref[...] = jnp.where(is1, 1.0 / denom,
                               jnp.where(iota == idx2, e2 / denom, 0.0))


@functools.partial(jax.jit, static_argnames=("block_t",))
def _noisy_topk_gating(x, w_gate, block_t=512):
    t, d = x.shape
    e = w_gate.shape[0]
    grid = (t // block_t,)
    gates, load = pl.pallas_call(
        _gating_block,
        grid=grid,
        in_specs=[
            pl.BlockSpec((block_t, d), lambda i: (i, 0)),
            pl.BlockSpec((e, d), lambda i: (0, 0)),
        ],
        out_specs=[
            pl.BlockSpec((block_t, e), lambda i: (i, 0)),
            pl.BlockSpec((block_t, e), lambda i: (i, 0)),
        ],
        out_shape=[
            jax.ShapeDtypeStruct((t, e), jnp.float32),
            jax.ShapeDtypeStruct((t, e), jnp.float32),
        ],
    )(x, w_gate)
    return gates, load


def kernel(x, W_gate, W_noise):
    del W_noise
    return _noisy_topk_gating(x, W_gate)
